# baseline (speedup 1.0000x reference)
import functools

import jax
import jax.numpy as jnp
from jax import lax
from jax.experimental import pallas as pl
from jax.experimental.pallas import tpu as pltpu
from jax.experimental.pallas import tpu_sc as plsc

_LANES = 16


def _newton_rsqrt(a):
    i = plsc.bitcast(a, jnp.int32)
    y = plsc.bitcast(jnp.int32(0x5F3759DF) - (i >> 1), jnp.float32)
    for _ in range(3):
        y = y * (1.5 - (0.5 * a) * y * y)
    return y


def _make_sc_gather(B, T_pad):
    mesh = plsc.VectorSubcoreMesh(core_axis_name="c", subcore_axis_name="s")
    n_chunks = B // _LANES
    f32 = jnp.float32

    @functools.partial(
        pl.kernel,
        out_type=(
            jax.ShapeDtypeStruct((B,), f32),
            jax.ShapeDtypeStruct((B,), f32),
        ),
        mesh=mesh,
        scratch_types=[
            pltpu.VMEM((T_pad,), f32),
            pltpu.VMEM((_LANES,), jnp.int32),
            pltpu.VMEM((_LANES,), f32),
            pltpu.VMEM((_LANES,), f32),
        ],
        compiler_params=pltpu.CompilerParams(needs_layout_passes=False),
    )
    def sc_gather(t_hbm, ab_hbm, c1_hbm, c2_hbm, table_v, t_v, c1_v, c2_v):
        w = lax.axis_index("s") * 2 + lax.axis_index("c")

        @pl.when(w < n_chunks)
        def _():
            base = w * _LANES
            pltpu.sync_copy(ab_hbm, table_v)
            pltpu.sync_copy(t_hbm.at[pl.ds(base, _LANES)], t_v)
            a = plsc.load_gather(table_v, [t_v[...]])
            c1_v[...] = _newton_rsqrt(a)
            c2_v[...] = _newton_rsqrt(a / (1.0 - a))
            pltpu.sync_copy(c1_v, c1_hbm.at[pl.ds(base, _LANES)])
            pltpu.sync_copy(c2_v, c2_hbm.at[pl.ds(base, _LANES)])

    return sc_gather


def _tc_combine_body(c1_ref, c2_ref, x_ref, n_ref, o_ref):
    b = pl.program_id(0)
    o_ref[...] = c1_ref[b] * x_ref[...] - c2_ref[b] * n_ref[...]


def kernel(x_t, t, pred_noise, alphas_bar):
    B, C, H, W = x_t.shape
    T = alphas_bar.shape[0]

    T_pad = (T + 255) // 256 * 256
    ab = jnp.concatenate([alphas_bar, jnp.ones((T_pad - T,), jnp.float32)])

    c1, c2 = _make_sc_gather(B, T_pad)(t, ab)

    N = C * H * W
    assert N % 128 == 0
    S = N // 128
    x2 = x_t.reshape(B, S, 128)
    n2 = pred_noise.reshape(B, S, 128)

    out = pl.pallas_call(
        _tc_combine_body,
        grid=(B,),
        in_specs=[
            pl.BlockSpec(memory_space=pltpu.SMEM),
            pl.BlockSpec(memory_space=pltpu.SMEM),
            pl.BlockSpec((1, S, 128), lambda b: (b, 0, 0)),
            pl.BlockSpec((1, S, 128), lambda b: (b, 0, 0)),
        ],
        out_specs=pl.BlockSpec((1, S, 128), lambda b: (b, 0, 0)),
        out_shape=jax.ShapeDtypeStruct((B, S, 128), jnp.float32),
    )(c1, c2, x2, n2)

    return out.reshape(B, C, H, W)


# speedup vs baseline: 1.0516x; 1.0516x over previous
import jax
import jax.numpy as jnp
from jax.experimental import pallas as pl
from jax.experimental.pallas import tpu as pltpu


def _body(t_ref, ab_ref, x_ref, n_ref, o_ref):
    b = pl.program_id(0)
    a = ab_ref[t_ref[b]]
    c1 = jax.lax.rsqrt(a)
    c2 = jnp.sqrt(1.0 / a - 1.0)
    o_ref[...] = c1 * x_ref[...] - c2 * n_ref[...]


def kernel(x_t, t, pred_noise, alphas_bar):
    B, C, H, W = x_t.shape
    N = C * H * W
    S = N // 128
    x2 = x_t.reshape(B, S, 128)
    n2 = pred_noise.reshape(B, S, 128)

    out = pl.pallas_call(
        _body,
        grid=(B,),
        in_specs=[
            pl.BlockSpec(memory_space=pltpu.SMEM),
            pl.BlockSpec(memory_space=pltpu.SMEM),
            pl.BlockSpec((1, S, 128), lambda b: (b, 0, 0)),
            pl.BlockSpec((1, S, 128), lambda b: (b, 0, 0)),
        ],
        out_specs=pl.BlockSpec((1, S, 128), lambda b: (b, 0, 0)),
        out_shape=jax.ShapeDtypeStruct((B, S, 128), jnp.float32),
    )(t, alphas_bar, x2, n2)

    return out.reshape(B, C, H, W)


# speedup vs baseline: 1.1711x; 1.1136x over previous
import jax
import jax.numpy as jnp
from jax.experimental import pallas as pl
from jax.experimental.pallas import tpu as pltpu

_R = 8


def _body(t_ref, ab_ref, x_ref, n_ref, o_ref):
    g = pl.program_id(0)
    for r in range(_R):
        a = ab_ref[t_ref[g * _R + r]]
        c1 = jax.lax.rsqrt(a)
        c2 = jnp.sqrt(1.0 / a - 1.0)
        o_ref[r] = c1 * x_ref[r] - c2 * n_ref[r]


def kernel(x_t, t, pred_noise, alphas_bar):
    B, C, H, W = x_t.shape
    N = C * H * W
    S = N // 128
    x2 = x_t.reshape(B, S, 128)
    n2 = pred_noise.reshape(B, S, 128)

    out = pl.pallas_call(
        _body,
        grid=(B // _R,),
        in_specs=[
            pl.BlockSpec(memory_space=pltpu.SMEM),
            pl.BlockSpec(memory_space=pltpu.SMEM),
            pl.BlockSpec((_R, S, 128), lambda b: (b, 0, 0)),
            pl.BlockSpec((_R, S, 128), lambda b: (b, 0, 0)),
        ],
        out_specs=pl.BlockSpec((_R, S, 128), lambda b: (b, 0, 0)),
        out_shape=jax.ShapeDtypeStruct((B, S, 128), jnp.float32),
    )(t, alphas_bar, x2, n2)

    return out.reshape(B, C, H, W)


# speedup vs baseline: 1.1778x; 1.0057x over previous
import jax
import jax.numpy as jnp
from jax import lax
from jax.experimental import pallas as pl
from jax.experimental.pallas import tpu as pltpu

_K = 8


def _coeffs(t_ref, ab_ref, i):
    a = ab_ref[t_ref[i]]
    return jax.lax.rsqrt(a), jnp.sqrt(1.0 / a - 1.0)


def _body(t_ref, ab_ref, x_hbm, n_hbm, o_hbm, xb, nb, ob, sx, sn, so):
    nch = pl.num_programs(0)
    i = pl.program_id(0)
    slot = lax.rem(i, _K)

    @pl.when(i == 0)
    def _prologue():
        for j in range(_K):
            pltpu.make_async_copy(x_hbm.at[j], xb.at[j], sx.at[j]).start()
            pltpu.make_async_copy(n_hbm.at[j], nb.at[j], sn.at[j]).start()

    pltpu.make_async_copy(x_hbm.at[i], xb.at[slot], sx.at[slot]).wait()
    pltpu.make_async_copy(n_hbm.at[i], nb.at[slot], sn.at[slot]).wait()

    @pl.when(i >= _K)
    def _drain_out():
        pltpu.make_async_copy(ob.at[slot], o_hbm.at[i - _K], so.at[slot]).wait()

    c1, c2 = _coeffs(t_ref, ab_ref, i)
    ob.at[slot][...] = c1 * xb.at[slot][...] - c2 * nb.at[slot][...]
    pltpu.make_async_copy(ob.at[slot], o_hbm.at[i], so.at[slot]).start()

    @pl.when(i + _K < nch)
    def _prefetch():
        pltpu.make_async_copy(x_hbm.at[i + _K], xb.at[slot], sx.at[slot]).start()
        pltpu.make_async_copy(n_hbm.at[i + _K], nb.at[slot], sn.at[slot]).start()

    @pl.when(i == nch - 1)
    def _epilogue():
        for j in range(_K):
            pltpu.make_async_copy(
                ob.at[j], o_hbm.at[nch - _K + j], so.at[j]
            ).wait()


def kernel(x_t, t, pred_noise, alphas_bar):
    B, C, H, W = x_t.shape
    N = C * H * W
    S = N // 128
    x2 = x_t.reshape(B, S, 128)
    n2 = pred_noise.reshape(B, S, 128)

    out = pl.pallas_call(
        _body,
        grid=(B,),
        in_specs=[
            pl.BlockSpec(memory_space=pltpu.SMEM),
            pl.BlockSpec(memory_space=pltpu.SMEM),
            pl.BlockSpec(memory_space=pltpu.MemorySpace.HBM),
            pl.BlockSpec(memory_space=pltpu.MemorySpace.HBM),
        ],
        out_specs=pl.BlockSpec(memory_space=pltpu.MemorySpace.HBM),
        out_shape=jax.ShapeDtypeStruct((B, S, 128), jnp.float32),
        scratch_shapes=[
            pltpu.VMEM((_K, S, 128), jnp.float32),
            pltpu.VMEM((_K, S, 128), jnp.float32),
            pltpu.VMEM((_K, S, 128), jnp.float32),
            pltpu.SemaphoreType.DMA((_K,)),
            pltpu.SemaphoreType.DMA((_K,)),
            pltpu.SemaphoreType.DMA((_K,)),
        ],
    )(t, alphas_bar, x2, n2)

    return out.reshape(B, C, H, W)


# speedup vs baseline: 4.9504x; 4.2030x over previous
import jax
import jax.numpy as jnp
from jax import lax
from jax.experimental import pallas as pl
from jax.experimental.pallas import tpu as pltpu

_K = 8


def _coeffs(t_ref, ab_ref, i):
    a = ab_ref[t_ref[i]]
    return jax.lax.rsqrt(a), jnp.sqrt(1.0 / a - 1.0)


def _body(t_ref, ab_ref, x_hbm, n_hbm, o_hbm, xb, nb, ob, sx, sn, so):
    nch = pl.num_programs(0)
    i = pl.program_id(0)
    slot = lax.rem(i, _K)

    @pl.when(i == 0)
    def _prologue():
        for j in range(_K):
            pltpu.make_async_copy(x_hbm.at[j], xb.at[j], sx.at[j]).start()
            pltpu.make_async_copy(n_hbm.at[j], nb.at[j], sn.at[j]).start()

    pltpu.make_async_copy(x_hbm.at[i], xb.at[slot], sx.at[slot]).wait()
    pltpu.make_async_copy(n_hbm.at[i], nb.at[slot], sn.at[slot]).wait()

    @pl.when(i >= _K)
    def _drain_out():
        pltpu.make_async_copy(ob.at[slot], o_hbm.at[i - _K], so.at[slot]).wait()

    c1, c2 = _coeffs(t_ref, ab_ref, i)
    ob.at[slot][...] = c1 * xb.at[slot][...] - c2 * nb.at[slot][...]
    pltpu.make_async_copy(ob.at[slot], o_hbm.at[i], so.at[slot]).start()

    @pl.when(i + _K < nch)
    def _prefetch():
        pltpu.make_async_copy(x_hbm.at[i + _K], xb.at[slot], sx.at[slot]).start()
        pltpu.make_async_copy(n_hbm.at[i + _K], nb.at[slot], sn.at[slot]).start()

    @pl.when(i == nch - 1)
    def _epilogue():
        for j in range(_K):
            pltpu.make_async_copy(
                ob.at[j], o_hbm.at[nch - _K + j], so.at[j]
            ).wait()


def kernel(x_t, t, pred_noise, alphas_bar):
    B, C, H, W = x_t.shape

    out = pl.pallas_call(
        _body,
        grid=(B,),
        in_specs=[
            pl.BlockSpec(memory_space=pltpu.SMEM),
            pl.BlockSpec(memory_space=pltpu.SMEM),
            pl.BlockSpec(memory_space=pltpu.MemorySpace.HBM),
            pl.BlockSpec(memory_space=pltpu.MemorySpace.HBM),
        ],
        out_specs=pl.BlockSpec(memory_space=pltpu.MemorySpace.HBM),
        out_shape=jax.ShapeDtypeStruct((B, C, H, W), jnp.float32),
        scratch_shapes=[
            pltpu.VMEM((_K, C, H, W), jnp.float32),
            pltpu.VMEM((_K, C, H, W), jnp.float32),
            pltpu.VMEM((_K, C, H, W), jnp.float32),
            pltpu.SemaphoreType.DMA((_K,)),
            pltpu.SemaphoreType.DMA((_K,)),
            pltpu.SemaphoreType.DMA((_K,)),
        ],
    )(t, alphas_bar, x_t, pred_noise)

    return out
